# Initial kernel scaffold; baseline (speedup 1.0000x reference)
import jax
import jax.numpy as jnp
from jax.experimental import pallas as pl


def kernel(text_encoded, mask, duration_target, W, b):
    raise NotImplementedError("write your pallas kernel here")


# speedup vs baseline: 5.8072x; 5.8072x over previous
import functools

import jax
import jax.numpy as jnp
from jax import lax
from jax.experimental import pallas as pl
from jax.experimental.pallas import tpu as pltpu
from jax.experimental.pallas import tpu_sc as plsc

B, T, D = 16, 512, 256
MAX_FRAMES = 4096
L = 16
NC, NS = 2, 16
HALF = MAX_FRAMES // 2
ROWS_CHUNK = 128
NCHUNK = HALF // ROWS_CHUNK
ZROW = B * T


_GATHER_DNUMS = lax.GatherDimensionNumbers(
    offset_dims=(), collapsed_slice_dims=(0,), start_index_map=(0,))


def _vtake(v, idx):
    return lax.gather(v, idx[:, None], _GATHER_DNUMS, (1,),
                      mode=lax.GatherScatterMode.PROMISE_IN_BOUNDS)


def _cummax16(v):
    iota = lax.iota(jnp.int32, L)
    for k in (1, 2, 4, 8):
        v = jnp.maximum(v, _vtake(v, jnp.maximum(iota - k, 0)))
    return v


def _cumsum16(v):
    iota = lax.iota(jnp.int32, L)
    for k in (1, 2, 4, 8):
        t = _vtake(v, jnp.maximum(iota - k, 0))
        v = v + jnp.where(iota >= k, t, 0)
    return v


def _vlast(v):
    return _vtake(v, jnp.full((L,), L - 1, jnp.int32))


def _sc_expand(dur, mask_i32, x_flat):
    mesh = plsc.VectorSubcoreMesh(core_axis_name="c", subcore_axis_name="s")

    @functools.partial(
        pl.kernel,
        mesh=mesh,
        compiler_params=pltpu.CompilerParams(needs_layout_passes=False),
        out_type=(
            jax.ShapeDtypeStruct((B * MAX_FRAMES, D), jnp.float32),
            jax.ShapeDtypeStruct((B, MAX_FRAMES), jnp.int32),
        ),
        scratch_types=[
            pltpu.VMEM((T,), jnp.int32),
            pltpu.VMEM((T,), jnp.int32),
            pltpu.VMEM((T,), jnp.int32),
            pltpu.VMEM((HALF,), jnp.int32),
            pltpu.VMEM((HALF,), jnp.int32),
            pltpu.VMEM((ROWS_CHUNK, D), jnp.float32),
            pltpu.VMEM((ROWS_CHUNK, D), jnp.float32),
            pltpu.SemaphoreType.DMA,
            pltpu.SemaphoreType.DMA,
        ],
    )
    def k(dur_hbm, mask_hbm, xflat_hbm, out_hbm, fm_hbm,
          dur_v, mask_v, cum_v, gidx_v, fm_v, rows0_v, rows1_v, sem0, sem1):
        c_ax = lax.axis_index("c")
        s_ax = lax.axis_index("s")
        b = s_ax
        h = c_ax

        pltpu.sync_copy(dur_hbm.at[b], dur_v)
        pltpu.sync_copy(mask_hbm.at[b], mask_v)

        def a_body(i, carry):
            d = dur_v[pl.ds(i * L, L)] * mask_v[pl.ds(i * L, L)]
            c = _cumsum16(d) + carry
            cum_v[pl.ds(i * L, L)] = c
            return _vlast(c)
        total = lax.fori_loop(0, T // L, a_body, jnp.zeros((L,), jnp.int32))

        nhalf = HALF // L

        def m_body(j, carry):
            fv = lax.iota(jnp.int32, L) + j * L
            pos = jnp.zeros((L,), jnp.int32)
            for w in (256, 128, 64, 32, 16, 8, 4, 2, 1):
                cmid = plsc.load_gather(cum_v, [pos + (w - 1)])
                pos = jnp.where(cmid <= fv, pos + w, pos)
            fm = fv < total
            gi = jnp.where(fm, b * T + jnp.minimum(pos, T - 1), ZROW)
            loc = (j - h * nhalf) * L
            gidx_v[pl.ds(loc, L)] = gi
            fm_v[pl.ds(loc, L)] = fm.astype(jnp.int32)
            return carry

        lax.fori_loop(h * nhalf, (h + 1) * nhalf, m_body, 0)

        pltpu.sync_copy(fm_v, fm_hbm.at[b, pl.ds(h * HALF, HALF)])

        row0 = b * MAX_FRAMES + h * HALF
        bufs = (rows0_v, rows1_v)
        sems = (sem0, sem1)
        pend = pltpu.async_copy(
            xflat_hbm.at[gidx_v.at[pl.ds(0, ROWS_CHUNK)]], bufs[0], sems[0])
        for c in range(NCHUNK):
            cur = bufs[c % 2]
            pend.wait()
            if c + 1 < NCHUNK:
                pend = pltpu.async_copy(
                    xflat_hbm.at[gidx_v.at[pl.ds((c + 1) * ROWS_CHUNK,
                                                 ROWS_CHUNK)]],
                    bufs[(c + 1) % 2], sems[(c + 1) % 2])
            pltpu.sync_copy(cur, out_hbm.at[pl.ds(row0 + c * ROWS_CHUNK,
                                                  ROWS_CHUNK)])

    return k(dur, mask_i32, x_flat)


def _pld_body(x_ref, w_ref, b_ref, o_ref):
    o_ref[...] = jnp.sum(x_ref[...] * w_ref[...][None, None, :], axis=2) + b_ref[0]


def _tc_pld(x, w_row, bias):
    return pl.pallas_call(
        _pld_body,
        grid=(2,),
        in_specs=[
            pl.BlockSpec((B // 2, T, D), lambda i: (i, 0, 0)),
            pl.BlockSpec((D,), lambda i: (0,)),
            pl.BlockSpec(memory_space=pltpu.SMEM),
        ],
        out_specs=pl.BlockSpec((B // 2, T), lambda i: (i, 0)),
        out_shape=jax.ShapeDtypeStruct((B, T), jnp.float32),
    )(x, w_row, bias)


@jax.jit
def kernel(text_encoded, mask, duration_target, W, b):
    pld = _tc_pld(text_encoded, W.reshape(-1), b)
    x_flat = jnp.concatenate(
        [text_encoded.reshape(B * T, D), jnp.zeros((8, D), jnp.float32)], axis=0)
    expanded_flat, fm_i32 = _sc_expand(
        duration_target.astype(jnp.int32), mask.astype(jnp.int32), x_flat)
    return (expanded_flat.reshape(B, MAX_FRAMES, D), pld, fm_i32.astype(bool))


# speedup vs baseline: 54.9383x; 9.4604x over previous
import functools

import jax
import jax.numpy as jnp
from jax import lax
from jax.experimental import pallas as pl
from jax.experimental.pallas import tpu as pltpu
from jax.experimental.pallas import tpu_sc as plsc

B, T, D = 16, 512, 256
MAX_FRAMES = 4096
L = 16
HALF = MAX_FRAMES // 2
ROWS_CHUNK = 32
NCHUNK = HALF // ROWS_CHUNK
BPC = 8
ZOFF = BPC * T

_GATHER_DNUMS = lax.GatherDimensionNumbers(
    offset_dims=(), collapsed_slice_dims=(0,), start_index_map=(0,))


def _vtake(v, idx):
    return lax.gather(v, idx[:, None], _GATHER_DNUMS, (1,),
                      mode=lax.GatherScatterMode.PROMISE_IN_BOUNDS)


def _cumsum16(v):
    iota = lax.iota(jnp.int32, L)
    for k in (1, 2, 4, 8):
        t = _vtake(v, jnp.maximum(iota - k, 0))
        v = v + jnp.where(iota >= k, t, 0)
    return v


def _vlast(v):
    return _vtake(v, jnp.full((L,), L - 1, jnp.int32))


def _sc_expand(dur, mask_i32, x_flat):
    mesh = plsc.VectorSubcoreMesh(core_axis_name="c", subcore_axis_name="s")

    @functools.partial(
        pl.kernel,
        mesh=mesh,
        compiler_params=pltpu.CompilerParams(
            needs_layout_passes=False, use_tc_tiling_on_sc=False),
        out_type=(
            jax.ShapeDtypeStruct((B * MAX_FRAMES, D), jnp.float32),
            jax.ShapeDtypeStruct((B, MAX_FRAMES), jnp.int32),
        ),
        scratch_types=[
            pltpu.VMEM_SHARED((BPC * T + 8, D), jnp.float32),
            pltpu.VMEM((T,), jnp.int32),
            pltpu.VMEM((T,), jnp.int32),
            pltpu.VMEM((T,), jnp.int32),
            pltpu.VMEM((HALF,), jnp.int32),
            pltpu.VMEM((HALF,), jnp.int32),
            pltpu.VMEM((8, D), jnp.float32),
            pltpu.VMEM((ROWS_CHUNK, D), jnp.float32),
            pltpu.VMEM((ROWS_CHUNK, D), jnp.float32),
            pltpu.SemaphoreType.DMA,
            pltpu.SemaphoreType.DMA,
            pltpu.SemaphoreType.DMA,
        ],
    )
    def k(dur_hbm, mask_hbm, xflat_hbm, out_hbm, fm_hbm,
          rows_sh, dur_v, mask_v, cum_v, gidx_v, fm_v, zrow_v,
          buf0_v, buf1_v, sem0, semo0, semo1):
        c_ax = lax.axis_index("c")
        s_ax = lax.axis_index("s")
        b_loc = s_ax // 2
        h = s_ax % 2
        b = c_ax * BPC + b_loc

        pltpu.sync_copy(xflat_hbm.at[pl.ds(b * T + h * (T // 2), T // 2)],
                        rows_sh.at[pl.ds(b_loc * T + h * (T // 2), T // 2)])

        @pl.when(s_ax == 0)
        def _publish_zero():
            def z_body(r, carry):
                zrow_v[r % 8, pl.ds((r // 8) * L, L)] = (
                    jnp.zeros((L,), jnp.float32))
                return carry
            lax.fori_loop(0, 8 * D // L, z_body, 0)
            pltpu.sync_copy(zrow_v, rows_sh.at[pl.ds(ZOFF, 8)])

        pltpu.sync_copy(dur_hbm.at[b], dur_v)
        pltpu.sync_copy(mask_hbm.at[b], mask_v)

        def a_body(i, carry):
            d = dur_v[pl.ds(i * L, L)] * mask_v[pl.ds(i * L, L)]
            c = _cumsum16(d) + carry
            cum_v[pl.ds(i * L, L)] = c
            return _vlast(c)
        total = lax.fori_loop(0, T // L, a_body, jnp.zeros((L,), jnp.int32))

        nhalf = HALF // L
        base = b_loc * T

        def m_body(j, carry):
            fv = lax.iota(jnp.int32, L) + j * L
            pos = jnp.zeros((L,), jnp.int32)
            for w in (256, 128, 64, 32, 16, 8, 4, 2, 1):
                cmid = plsc.load_gather(cum_v, [pos + (w - 1)])
                pos = jnp.where(cmid <= fv, pos + w, pos)
            fm = fv < total
            gi = jnp.where(fm, base + jnp.minimum(pos, T - 1), ZOFF)
            loc = (j - h * nhalf) * L
            gidx_v[pl.ds(loc, L)] = gi
            fm_v[pl.ds(loc, L)] = fm.astype(jnp.int32)
            return carry

        lax.fori_loop(h * nhalf, (h + 1) * nhalf, m_body, 0)

        pltpu.sync_copy(fm_v, fm_hbm.at[b, pl.ds(h * HALF, HALF)])

        plsc.subcore_barrier()

        row0 = b * MAX_FRAMES + h * HALF
        bufs = (buf0_v, buf1_v)
        sems = (semo0, semo1)
        pend = [None, None]
        gpc = ROWS_CHUNK // L

        for c in range(NCHUNK):
            buf = bufs[c % 2]
            if pend[c % 2] is not None:
                pend[c % 2].wait()

            def g_body(g, carry, buf=buf, c=c):
                vec = gidx_v[pl.ds((c * gpc + g) * L, L)]
                for jj in range(L):
                    pltpu.async_copy(rows_sh.at[pl.ds(vec[jj], 1)],
                                     buf.at[pl.ds(g * L + jj, 1)], sem0)
                return carry

            lax.fori_loop(0, gpc, g_body, 0)
            pltpu.make_async_copy(
                xflat_hbm.at[pl.ds(0, ROWS_CHUNK)], buf, sem0).wait()
            pend[c % 2] = pltpu.async_copy(
                buf, out_hbm.at[pl.ds(row0 + c * ROWS_CHUNK, ROWS_CHUNK)],
                sems[c % 2])
        pend[0].wait()
        pend[1].wait()

    return k(dur, mask_i32, x_flat)


def _pld_body(x_ref, w_ref, b_ref, o_ref):
    o_ref[...] = jnp.sum(x_ref[...] * w_ref[...][None, None, :], axis=2) + b_ref[0]


def _tc_pld(x, w_row, bias):
    return pl.pallas_call(
        _pld_body,
        grid=(2,),
        in_specs=[
            pl.BlockSpec((B // 2, T, D), lambda i: (i, 0, 0)),
            pl.BlockSpec((D,), lambda i: (0,)),
            pl.BlockSpec(memory_space=pltpu.SMEM),
        ],
        out_specs=pl.BlockSpec((B // 2, T), lambda i: (i, 0)),
        out_shape=jax.ShapeDtypeStruct((B, T), jnp.float32),
    )(x, w_row, bias)


@jax.jit
def kernel(text_encoded, mask, duration_target, W, b):
    pld = _tc_pld(text_encoded, W.reshape(-1), b)
    x_flat = text_encoded.reshape(B * T, D)
    expanded_flat, fm_i32 = _sc_expand(
        duration_target.astype(jnp.int32), mask.astype(jnp.int32), x_flat)
    return (expanded_flat.reshape(B, MAX_FRAMES, D), pld, fm_i32.astype(bool))
